# baseline (speedup 1.0000x reference)
import functools

import jax
import jax.numpy as jnp
from jax import lax
from jax.experimental import pallas as pl
from jax.experimental.pallas import tpu as pltpu
from jax.experimental.pallas import tpu_sc as plsc

NC = 2
NS = 16
BATCH = 125
KI = 8
KI_DEG = 8

_mesh = plsc.VectorSubcoreMesh(
    core_axis_name="c", subcore_axis_name="s", num_cores=NC, num_subcores=NS)
_sc_params = pltpu.CompilerParams(use_tc_tiling_on_sc=False)


def _deg_body(ei3, zeros8, ones8, out, idx, ones_v, acc, ssem):
    c = lax.axis_index("c")
    s = lax.axis_index("s")
    npad = acc.shape[0]
    tchunk = npad // NS
    rows_half = ei3.shape[1] // NC
    rpt = rows_half // NS
    nblk = rpt // KI_DEG
    pltpu.sync_copy(zeros8.at[pl.ds(s * tchunk, tchunk)],
                    acc.at[pl.ds(s * tchunk, tchunk)])
    pltpu.sync_copy(ones8, ones_v)
    plsc.subcore_barrier()
    row0 = c * rows_half + s * rpt

    def drain_scatters(p):
        for j in range(KI_DEG):
            pltpu.make_async_copy(ones_v, acc.at[idx.at[p].at[j]],
                                  ssem).wait()

    def outer(i2, carry):
        for p in range(2):
            blk = i2 * 2 + p

            @pl.when(blk > 0)
            def _():
                drain_scatters(1 - p)

            pltpu.sync_copy(
                ei3.at[1, pl.ds(row0 + blk * KI_DEG, KI_DEG)], idx.at[p])
            for j in range(KI_DEG):
                pltpu.async_copy(ones_v, acc.at[idx.at[p].at[j]], ssem,
                                 add=True)
        return carry

    lax.fori_loop(0, nblk // 2, outer, 0)
    drain_scatters((nblk - 1) % 2)
    plsc.subcore_barrier()
    pltpu.sync_copy(acc.at[pl.ds(s * tchunk, tchunk)],
                    out.at[c, pl.ds(s * tchunk, tchunk)])


def _agg_body(ei3, ys2, zeros24, out, sidx, didx, rows, acc, gsem, ssem):
    c = lax.axis_index("c")
    s = lax.axis_index("s")
    npad = acc.shape[0]
    tchunk = npad // NS
    nv = ys2.shape[0] // NC
    rows_tot = ei3.shape[1]
    rpt = rows_tot // NS
    nblk = rpt // KI
    table = ys2.at[pl.ds(c * nv, nv)]
    pltpu.sync_copy(zeros24.at[pl.ds(s * tchunk, tchunk)],
                    acc.at[pl.ds(s * tchunk, tchunk)])
    plsc.subcore_barrier()
    row0 = s * rpt

    def load_idx(blk, p):
        pltpu.sync_copy(ei3.at[0, pl.ds(row0 + blk * KI, KI)], sidx.at[p])
        pltpu.sync_copy(ei3.at[1, pl.ds(row0 + blk * KI, KI)], didx.at[p])

    def fire_gathers(p):
        for j in range(KI):
            pltpu.async_copy(table.at[sidx.at[p].at[j]], rows.at[p].at[j],
                             gsem)

    def drain_gathers(p):
        for j in range(KI):
            pltpu.make_async_copy(table.at[sidx.at[p].at[j]],
                                  rows.at[p].at[j], gsem).wait()

    def fire_scatters(p):
        for j in range(KI):
            pltpu.async_copy(rows.at[p].at[j], acc.at[didx.at[p].at[j]],
                             ssem, add=True)

    def drain_scatters(p):
        for j in range(KI):
            pltpu.make_async_copy(rows.at[p].at[j],
                                  acc.at[didx.at[p].at[j]], ssem).wait()

    load_idx(0, 0)
    fire_gathers(0)

    def outer(i2, carry):
        for p in range(2):
            blk = i2 * 2 + p
            drain_gathers(p)
            fire_scatters(p)

            @pl.when(blk > 0)
            def _():
                drain_scatters(1 - p)

            @pl.when(blk + 1 < nblk)
            def _():
                load_idx(blk + 1, 1 - p)
                fire_gathers(1 - p)
        return carry

    lax.fori_loop(0, nblk // 2, outer, 0)
    drain_scatters((nblk - 1) % 2)
    plsc.subcore_barrier()
    pltpu.sync_copy(acc.at[pl.ds(s * tchunk, tchunk)],
                    out.at[c, pl.ds(s * tchunk, tchunk)])


def _proj_body(xs_ref, degn_ref, m_ref, ys_ref, dinv_ref):
    d = degn_ref[0, :, 0] + degn_ref[1, :, 0] + 1.0
    dinv = lax.rsqrt(d)
    dinv_ref[:, 0] = dinv
    xb = xs_ref[0]
    m = m_ref[...]
    y = jnp.dot(xb, m, precision=lax.Precision.HIGHEST)
    ys_ref[...] = y * dinv[:, None]


def _final_body(agg_ref, ys0_ref, ys1_ref, dinv_ref, attw_ref, p_ref,
                out_ref):
    dinv = dinv_ref[:, 0:1]
    aw = attw_ref[...]
    m = jnp.max(aw)
    e = jnp.exp(aw - m)
    sm = e / jnp.sum(e)
    b_z = p_ref[0]
    b_h = p_ref[1]
    az = p_ref[2]
    cz = p_ref[3]
    ah = p_ref[4]
    ch = p_ref[5]
    t12 = aw.shape[1]
    for b, ysb in ((0, ys0_ref), (1, ys1_ref)):
        ab = agg_ref[b]
        yb = ysb[...]
        gz = dinv * (ab[:, 0:t12] + yb[:, 0:t12]) + b_z
        gh = dinv * (ab[:, t12:] + yb[:, t12:]) + b_h
        z = jax.nn.sigmoid(az * gz + cz)
        ht = jnp.tanh(ah * gh + ch)
        hb = jnp.sum((1.0 - z) * ht * sm, axis=1)
        out_ref[:, b] = jax.nn.sigmoid(jnp.maximum(hb, 0.0))


def kernel(x, edge_index, att, W_z, b_z, W_r, b_r, W_h, b_h,
           Wl_z, bl_z, Wl_r, bl_r, Wl_h, bl_h):
    B_, N, F, T = x.shape
    E = edge_index.shape[1]
    C = B_ * T
    f32 = jnp.float32

    rows = E // BATCH
    assert rows * BATCH == E and rows % (NC * NS * 2 * KI_DEG) == 0
    NPAD = -(-(N + 1) // (NS * 8)) * (NS * 8)

    ei3 = edge_index.reshape(2, rows, BATCH)

    zeros8 = jnp.zeros((NPAD, 8), f32)
    ones8 = jnp.ones((BATCH, 8), f32)
    zeros24 = jnp.zeros((NPAD, C), f32)

    deg_call = functools.partial(
        pl.kernel,
        out_type=jax.ShapeDtypeStruct((NC, NPAD, 8), f32),
        mesh=_mesh,
        compiler_params=_sc_params,
        scratch_types=[
            pltpu.VMEM((2, KI_DEG, BATCH), jnp.int32),
            pltpu.VMEM((BATCH, 8), f32),
            pltpu.VMEM_SHARED((NPAD, 8), f32),
            pltpu.SemaphoreType.DMA,
        ])(_deg_body)
    degn = deg_call(ei3, zeros8, ones8)

    xs = x.reshape(B_, N, F * T)
    wcat = jnp.concatenate([W_z, W_h], axis=1)
    mproj = jnp.kron(wcat, jnp.eye(T, dtype=f32))

    BN = 2000
    gridn = N // BN
    assert gridn * BN == N
    ys2, dinv2 = pl.pallas_call(
        _proj_body,
        grid=(B_, gridn),
        in_specs=[
            pl.BlockSpec((1, BN, F * T), lambda b, i: (b, i, 0)),
            pl.BlockSpec((NC, BN, 8), lambda b, i: (0, i, 0)),
            pl.BlockSpec((F * T, C), lambda b, i: (0, 0)),
        ],
        out_specs=[
            pl.BlockSpec((BN, C), lambda b, i: (b * (N // BN) + i, 0)),
            pl.BlockSpec((BN, 1), lambda b, i: (b * (N // BN) + i, 0)),
        ],
        out_shape=[
            jax.ShapeDtypeStruct((B_ * N, C), f32),
            jax.ShapeDtypeStruct((B_ * N, 1), f32),
        ],
    )(xs, degn, mproj)

    agg_call = functools.partial(
        pl.kernel,
        out_type=jax.ShapeDtypeStruct((NC, NPAD, C), f32),
        mesh=_mesh,
        compiler_params=_sc_params,
        scratch_types=[
            pltpu.VMEM((2, KI, BATCH), jnp.int32),
            pltpu.VMEM((2, KI, BATCH), jnp.int32),
            pltpu.VMEM((2, KI, BATCH, C), f32),
            pltpu.VMEM_SHARED((NPAD, C), f32),
            pltpu.SemaphoreType.DMA,
            pltpu.SemaphoreType.DMA,
        ])(_agg_body)
    aggn = agg_call(ei3, ys2, zeros24)

    attw = att.reshape(1, T)
    pvec = jnp.stack([b_z[0], b_h[0], Wl_z[0, 0], bl_z[0],
                      Wl_h[0, 0], bl_h[0]])

    outn = pl.pallas_call(
        _final_body,
        grid=(gridn,),
        in_specs=[
            pl.BlockSpec((NC, BN, C), lambda i: (0, i, 0)),
            pl.BlockSpec((BN, C), lambda i: (i, 0)),
            pl.BlockSpec((BN, C), lambda i: (N // BN + i, 0)),
            pl.BlockSpec((BN, 1), lambda i: (i, 0)),
            pl.BlockSpec((1, T), lambda i: (0, 0)),
            pl.BlockSpec(memory_space=pltpu.SMEM),
        ],
        out_specs=pl.BlockSpec((BN, 2), lambda i: (i, 0)),
        out_shape=jax.ShapeDtypeStruct((N, 2), f32),
    )(aggn, ys2, ys2, dinv2, attw, pvec)
    return outn.T


# speedup vs baseline: 1.0343x; 1.0343x over previous
import functools

import jax
import jax.numpy as jnp
from jax import lax
from jax.experimental import pallas as pl
from jax.experimental.pallas import tpu as pltpu
from jax.experimental.pallas import tpu_sc as plsc

NC = 2
NS = 16
BATCH = 125
KI = 8
KI_DEG = 8

_mesh = plsc.VectorSubcoreMesh(
    core_axis_name="c", subcore_axis_name="s", num_cores=NC, num_subcores=NS)
_sc_params = pltpu.CompilerParams(use_tc_tiling_on_sc=False)


def _deg_body(ei3, zeros8, ones8, out, idx, ones_v, acc, ssem):
    c = lax.axis_index("c")
    s = lax.axis_index("s")
    npad = acc.shape[0]
    tchunk = npad // NS
    rows_half = ei3.shape[1] // NC
    rpt = rows_half // NS
    nblk = rpt // KI_DEG
    pltpu.sync_copy(zeros8.at[pl.ds(s * tchunk, tchunk)],
                    acc.at[pl.ds(s * tchunk, tchunk)])
    pltpu.sync_copy(ones8, ones_v)
    plsc.subcore_barrier()
    row0 = c * rows_half + s * rpt

    def drain_scatters(p):
        for j in range(KI_DEG):
            pltpu.make_async_copy(ones_v, acc.at[idx.at[p].at[j]],
                                  ssem).wait()

    def outer(i2, carry):
        for p in range(2):
            blk = i2 * 2 + p

            @pl.when(blk > 0)
            def _():
                drain_scatters(1 - p)

            pltpu.sync_copy(
                ei3.at[1, pl.ds(row0 + blk * KI_DEG, KI_DEG)], idx.at[p])
            for j in range(KI_DEG):
                pltpu.async_copy(ones_v, acc.at[idx.at[p].at[j]], ssem,
                                 add=True)
        return carry

    lax.fori_loop(0, nblk // 2, outer, 0)
    drain_scatters((nblk - 1) % 2)
    plsc.subcore_barrier()
    pltpu.sync_copy(acc.at[pl.ds(s * tchunk, tchunk)],
                    out.at[pl.ds(c * npad + s * tchunk, tchunk)])


def _agg_body(ei3, ys2, zeros24, out, sidx, didx, rows, acc, gsem, ssem):
    c = lax.axis_index("c")
    s = lax.axis_index("s")
    npad = acc.shape[0]
    tchunk = npad // NS
    nv = ys2.shape[0] // NC
    rows_tot = ei3.shape[1]
    rpt = rows_tot // NS
    nblk = rpt // KI
    table = ys2.at[pl.ds(c * nv, nv)]
    pltpu.sync_copy(zeros24.at[pl.ds(s * tchunk, tchunk)],
                    acc.at[pl.ds(s * tchunk, tchunk)])
    plsc.subcore_barrier()
    row0 = s * rpt

    def load_idx(blk, p):
        pltpu.sync_copy(ei3.at[0, pl.ds(row0 + blk * KI, KI)], sidx.at[p])
        pltpu.sync_copy(ei3.at[1, pl.ds(row0 + blk * KI, KI)], didx.at[p])

    def fire_gathers(p):
        for j in range(KI):
            pltpu.async_copy(table.at[sidx.at[p].at[j]], rows.at[p].at[j],
                             gsem)

    def drain_gathers(p):
        for j in range(KI):
            pltpu.make_async_copy(table.at[sidx.at[p].at[j]],
                                  rows.at[p].at[j], gsem).wait()

    def fire_scatters(p):
        for j in range(KI):
            pltpu.async_copy(rows.at[p].at[j], acc.at[didx.at[p].at[j]],
                             ssem, add=True)

    def drain_scatters(p):
        for j in range(KI):
            pltpu.make_async_copy(rows.at[p].at[j],
                                  acc.at[didx.at[p].at[j]], ssem).wait()

    load_idx(0, 0)
    fire_gathers(0)

    def outer(i2, carry):
        for p in range(2):
            blk = i2 * 2 + p
            drain_gathers(p)
            fire_scatters(p)

            @pl.when(blk > 0)
            def _():
                drain_scatters(1 - p)

            @pl.when(blk + 1 < nblk)
            def _():
                load_idx(blk + 1, 1 - p)
                fire_gathers(1 - p)
        return carry

    lax.fori_loop(0, nblk // 2, outer, 0)
    drain_scatters((nblk - 1) % 2)
    plsc.subcore_barrier()
    pltpu.sync_copy(acc.at[pl.ds(s * tchunk, tchunk)],
                    out.at[pl.ds(c * npad + s * tchunk, tchunk)])


def _proj_body(xs_ref, degn_ref, m_ref, ys_ref, dinv_ref):
    d = degn_ref[0, :, 0] + degn_ref[1, :, 0] + 1.0
    dinv = lax.rsqrt(d)
    dinv_ref[:, 0] = dinv
    xb = xs_ref[...]
    m = m_ref[...]
    t12 = m.shape[1] // 2
    y0 = jnp.dot(xb[0], m, precision=lax.Precision.HIGHEST)
    y1 = jnp.dot(xb[1], m, precision=lax.Precision.HIGHEST)
    sc = dinv[:, None]
    ys_ref[0, :, :] = jnp.concatenate(
        [y0[:, 0:t12], y1[:, 0:t12]], axis=1) * sc
    ys_ref[1, :, :] = jnp.concatenate(
        [y0[:, t12:], y1[:, t12:]], axis=1) * sc


def _final_body(agg_ref, ys_ref, dinv_ref, attw_ref, p_ref, out_ref):
    dinv = dinv_ref[:, 0:1]
    aw = attw_ref[...]
    m = jnp.max(aw)
    e = jnp.exp(aw - m)
    sm = e / jnp.sum(e)
    b_z = p_ref[0]
    b_h = p_ref[1]
    az = p_ref[2]
    cz = p_ref[3]
    ah = p_ref[4]
    ch = p_ref[5]
    gz = dinv * (agg_ref[0] + ys_ref[0]) + b_z
    gh = dinv * (agg_ref[1] + ys_ref[1]) + b_h
    z = jax.nn.sigmoid(az * gz + cz)
    ht = jnp.tanh(ah * gh + ch)
    step = (1.0 - z) * ht * sm
    t12 = step.shape[1] // 2
    h0 = 2.0 * jnp.sum(step[:, 0:t12], axis=1)
    h1 = 2.0 * jnp.sum(step[:, t12:], axis=1)
    out_ref[:, 0] = jax.nn.sigmoid(jnp.maximum(h0, 0.0))
    out_ref[:, 1] = jax.nn.sigmoid(jnp.maximum(h1, 0.0))


def kernel(x, edge_index, att, W_z, b_z, W_r, b_r, W_h, b_h,
           Wl_z, bl_z, Wl_r, bl_r, Wl_h, bl_h):
    B_, N, F, T = x.shape
    E = edge_index.shape[1]
    C = B_ * T
    f32 = jnp.float32

    rows = E // BATCH
    assert rows * BATCH == E and rows % (NC * NS * 2 * KI_DEG) == 0
    NPAD = -(-(N + 1) // (NS * 8)) * (NS * 8)

    ei3 = edge_index.reshape(2, rows, BATCH)

    zeros8 = jnp.zeros((NPAD, 8), f32)
    ones8 = jnp.ones((BATCH, 8), f32)
    zeros24 = jnp.zeros((NPAD, C), f32)

    deg_call = functools.partial(
        pl.kernel,
        out_type=jax.ShapeDtypeStruct((NC * NPAD, 8), f32),
        mesh=_mesh,
        compiler_params=_sc_params,
        scratch_types=[
            pltpu.VMEM((2, KI_DEG, BATCH), jnp.int32),
            pltpu.VMEM((BATCH, 8), f32),
            pltpu.VMEM_SHARED((NPAD, 8), f32),
            pltpu.SemaphoreType.DMA,
        ])(_deg_body)
    deg8 = deg_call(ei3, zeros8, ones8)
    degn = deg8.reshape(NC, NPAD, 8)

    xs = x.reshape(B_, N, F * T)
    wcat = jnp.concatenate([W_z, W_h], axis=1)
    mproj = jnp.kron(wcat, jnp.eye(T, dtype=f32))

    BN = 2048
    grid = -(-N // BN)
    ys, dinv = pl.pallas_call(
        _proj_body,
        grid=(grid,),
        in_specs=[
            pl.BlockSpec((B_, BN, F * T), lambda i: (0, i, 0)),
            pl.BlockSpec((NC, BN, 8), lambda i: (0, i, 0)),
            pl.BlockSpec((F * T, C), lambda i: (0, 0)),
        ],
        out_specs=[
            pl.BlockSpec((2, BN, C), lambda i: (0, i, 0)),
            pl.BlockSpec((BN, 1), lambda i: (i, 0)),
        ],
        out_shape=[
            jax.ShapeDtypeStruct((2, N, C), f32),
            jax.ShapeDtypeStruct((N, 1), f32),
        ],
    )(xs, degn, mproj)

    ys2 = ys.reshape(2 * N, C)

    agg_call = functools.partial(
        pl.kernel,
        out_type=jax.ShapeDtypeStruct((NC * NPAD, C), f32),
        mesh=_mesh,
        compiler_params=_sc_params,
        scratch_types=[
            pltpu.VMEM((2, KI, BATCH), jnp.int32),
            pltpu.VMEM((2, KI, BATCH), jnp.int32),
            pltpu.VMEM((2, KI, BATCH, C), f32),
            pltpu.VMEM_SHARED((NPAD, C), f32),
            pltpu.SemaphoreType.DMA,
            pltpu.SemaphoreType.DMA,
        ])(_agg_body)
    agg = agg_call(ei3, ys2, zeros24)
    aggn = agg.reshape(NC, NPAD, C)

    attw = jnp.concatenate([att, att]).reshape(1, C)
    pvec = jnp.stack([b_z[0], b_h[0], Wl_z[0, 0], bl_z[0],
                      Wl_h[0, 0], bl_h[0]])

    outn = pl.pallas_call(
        _final_body,
        grid=(grid,),
        in_specs=[
            pl.BlockSpec((NC, BN, C), lambda i: (0, i, 0)),
            pl.BlockSpec((2, BN, C), lambda i: (0, i, 0)),
            pl.BlockSpec((BN, 1), lambda i: (i, 0)),
            pl.BlockSpec((1, C), lambda i: (0, 0)),
            pl.BlockSpec(memory_space=pltpu.SMEM),
        ],
        out_specs=pl.BlockSpec((BN, 2), lambda i: (i, 0)),
        out_shape=jax.ShapeDtypeStruct((N, 2), f32),
    )(aggn, ys, dinv, attw, pvec)
    return outn.T
